# Initial kernel scaffold; baseline (speedup 1.0000x reference)
import jax
import jax.numpy as jnp
from jax.experimental import pallas as pl


def kernel(boxes, scores, labels):
    raise NotImplementedError("write your pallas kernel here")


# speedup vs baseline: 66.3207x; 66.3207x over previous
import jax
import jax.numpy as jnp
from jax.experimental import pallas as pl
from jax.experimental.pallas import tpu as pltpu

N = 5000
T = 256
NP = 5120
NB = NP // T
IOU_THR = 0.6
SCORE_THR = 0.1


def _nms_kernel(ar_ref, ac_ref, out_ref, srr, scc, krr):
    f32 = jnp.float32
    m = jnp.max(ar_ref[0:4, :]) + 1.0
    offr = ar_ref[4:5, :] * m
    offc = ac_ref[:, 4:5] * m
    for c in range(4):
        srr[c:c + 1, :] = ar_ref[c:c + 1, :] + offr
        scc[:, c:c + 1] = ac_ref[:, c:c + 1] + offc
    srr[4:5, :] = (srr[2:3, :] - srr[0:1, :]) * (srr[3:4, :] - srr[1:2, :])
    scc[:, 4:5] = (scc[:, 2:3] - scc[:, 0:1]) * (scc[:, 3:4] - scc[:, 1:2])
    krr[0:1, :] = jnp.zeros((1, NP), f32)

    def sup_tile(sb, qb):
        px1 = scc[pl.ds(sb, T), 0:1]
        py1 = scc[pl.ds(sb, T), 1:2]
        px2 = scc[pl.ds(sb, T), 2:3]
        py2 = scc[pl.ds(sb, T), 3:4]
        pa = scc[pl.ds(sb, T), 4:5]
        qx1 = srr[0:1, pl.ds(qb, T)]
        qy1 = srr[1:2, pl.ds(qb, T)]
        qx2 = srr[2:3, pl.ds(qb, T)]
        qy2 = srr[3:4, pl.ds(qb, T)]
        qa = srr[4:5, pl.ds(qb, T)]
        iw = jnp.maximum(jnp.minimum(px2, qx2) - jnp.maximum(px1, qx1), 0.0)
        ih = jnp.maximum(jnp.minimum(py2, qy2) - jnp.maximum(py1, qy1), 0.0)
        inter = iw * ih
        union = pa + qa - inter
        return jnp.where(inter > IOU_THR * union, 1.0, 0.0)

    dot_dims = (((1,), (0,)), ((), ()))

    def block_body(j, _):
        base = j * T
        def strip(i, acc):
            sb = i * T
            sup = sup_tile(sb, base)
            kv = krr[0:1, pl.ds(sb, T)]
            return acc + jax.lax.dot_general(
                kv, sup, dot_dims, preferred_element_type=f32)
        acc = jax.lax.fori_loop(0, j, strip, jnp.zeros((1, T), f32))
        vb = jnp.where(ar_ref[5:6, pl.ds(base, T)] >= SCORE_THR, 1.0, 0.0)
        alive = jnp.where(acc > 0.5, 0.0, vb)
        sd = sup_tile(base, base)
        rowi = jax.lax.broadcasted_iota(jnp.int32, (T, T), 0)
        coli = jax.lax.broadcasted_iota(jnp.int32, (T, T), 1)
        sd = jnp.where(coli > rowi, sd, 0.0)
        krr[0:1, pl.ds(base, T)] = alive

        def fcond(c):
            it, ch = c
            return jnp.logical_and(ch, it < T + 8)

        def fbody(c):
            it, _ = c
            kb = krr[0:1, pl.ds(base, T)]
            supv = jax.lax.dot_general(kb, sd, dot_dims,
                                       preferred_element_type=f32)
            new = jnp.where(supv > 0.5, 0.0, alive)
            krr[0:1, pl.ds(base, T)] = new
            ch = jnp.sum(jnp.abs(new - kb)) > 0.0
            return (it + jnp.int32(1), ch)

        jax.lax.while_loop(fcond, fbody, (jnp.int32(0), jnp.bool_(True)))
        return 0

    jax.lax.fori_loop(0, NB, block_body, 0)
    out_ref[0:1, :] = krr[0:1, :] * ar_ref[5:6, :]


def _run_nms(ar, ac):
    return pl.pallas_call(
        _nms_kernel,
        out_shape=jax.ShapeDtypeStruct((1, NP), jnp.float32),
        scratch_shapes=[pltpu.VMEM((8, NP), jnp.float32),
                        pltpu.VMEM((NP, 8), jnp.float32),
                        pltpu.VMEM((1, NP), jnp.float32)],
    )(ar, ac)


def kernel(boxes, scores, labels):
    order = jnp.argsort(-scores)
    bs = boxes[order]
    ss = scores[order]
    ls = labels[order].astype(jnp.float32)
    pad = NP - N
    bsp = jnp.pad(bs, ((0, pad), (0, 0)))
    ssp = jnp.pad(ss, (0, pad), constant_values=-1.0)
    lsp = jnp.pad(ls, (0, pad))
    ac = jnp.concatenate(
        [bsp, lsp[:, None], ssp[:, None], jnp.zeros((NP, 2), jnp.float32)],
        axis=1)
    ar = ac.T
    out = _run_nms(ar, ac)
    return out[0, :N]
